# baseline (speedup 1.0000x reference)
import functools

import jax
import jax.numpy as jnp
from jax import lax
from jax.experimental import pallas as pl
from jax.experimental.pallas import tpu as pltpu
from jax.experimental.pallas import tpu_sc as plsc

N_NODES = 10000
N_EDGES = 320000
NC = 2
NS = 16
NW = NC * NS
CHUNK = 96
EPW_RAW = N_EDGES // NW
EPW = 10080
PAD = EPW - EPW_RAW
NCHUNKS = EPW // CHUNK
NRB = 3
PAD_ROWS = 8


STRIPE = 624


def _segment_sum_sc(x, edge_flat):
    n, d = x.shape
    n_acc = n + PAD_ROWS
    mesh = plsc.VectorSubcoreMesh(core_axis_name="c", subcore_axis_name="s")
    zeros = jnp.zeros((n_acc, d), jnp.float32)

    @functools.partial(
        pl.kernel,
        out_type=jax.ShapeDtypeStruct((NC, n, d), jnp.float32),
        mesh=mesh,
        scratch_types=(
            [pltpu.VMEM((EPW,), jnp.int32)]
            + [pltpu.VMEM((CHUNK,), jnp.int32) for _ in range(NRB)]
            + [pltpu.VMEM((CHUNK, d), jnp.float32) for _ in range(NRB)]
            + [pltpu.VMEM_SHARED((n_acc, d), jnp.float32)]
            + [pltpu.SemaphoreType.DMA for _ in range(2 * NRB)]
        ),
    )
    def seg_sum(x_hbm, ei_hbm, zeros_hbm, out_hbm, *refs):
        sidx_all = refs[0]
        didx = refs[1:1 + NRB]
        rows = refs[1 + NRB:1 + 2 * NRB]
        acc = refs[1 + 2 * NRB]
        sems = refs[2 + 2 * NRB:]
        dsem = sems[0:NRB]
        gsem = sems[NRB:2 * NRB]

        cid = lax.axis_index("c")
        sid = lax.axis_index("s")
        wid = sid * NC + cid
        base = wid * EPW

        def start_chunk(g, b):
            pltpu.make_async_copy(
                ei_hbm.at[pl.ds(NW * EPW + base + g * CHUNK, CHUNK)], didx[b], dsem[b]
            ).start()
            pltpu.make_async_copy(
                x_hbm.at[sidx_all.at[pl.ds(g * CHUNK, CHUNK)]], rows[b], gsem[b]
            ).start()

        @pl.when(sid < NS - 1)
        def _():
            pltpu.sync_copy(zeros_hbm.at[pl.ds(sid * STRIPE, STRIPE)],
                            acc.at[pl.ds(sid * STRIPE, STRIPE)])

        @pl.when(sid == NS - 1)
        def _():
            pltpu.sync_copy(zeros_hbm.at[pl.ds((NS - 1) * STRIPE,
                                               n_acc - (NS - 1) * STRIPE)],
                            acc.at[pl.ds((NS - 1) * STRIPE,
                                         n_acc - (NS - 1) * STRIPE)])

        pltpu.sync_copy(ei_hbm.at[pl.ds(base, EPW)], sidx_all)
        plsc.subcore_barrier()

        start_chunk(0, 0)
        start_chunk(1, 1)

        nslots = NRB * ((NCHUNKS + NRB - 1) // NRB)

        @pl.loop(0, nslots, step=NRB)
        def _(i):
            for u in range(NRB):
                c = i + u

                @pl.when(c < NCHUNKS)
                def _():
                    pltpu.make_async_copy(
                        ei_hbm.at[pl.ds(base, CHUNK)], didx[u], dsem[u]
                    ).wait()
                    pltpu.make_async_copy(
                        x_hbm.at[pl.ds(0, CHUNK)], rows[u], gsem[u]
                    ).wait()

                    @pl.when(c + 2 < NCHUNKS)
                    def _():
                        start_chunk(c + 2, (u + 2) % NRB)

                    pltpu.sync_copy(rows[u], acc.at[didx[u]], add=True)

        plsc.subcore_barrier()

        @pl.when(sid < NS - 1)
        def _():
            pltpu.sync_copy(acc.at[pl.ds(sid * STRIPE, STRIPE)],
                            out_hbm.at[cid, pl.ds(sid * STRIPE, STRIPE)])

        @pl.when(sid == NS - 1)
        def _():
            pltpu.sync_copy(acc.at[pl.ds((NS - 1) * STRIPE,
                                         n - (NS - 1) * STRIPE)],
                            out_hbm.at[cid, pl.ds((NS - 1) * STRIPE,
                                                  n - (NS - 1) * STRIPE)])

    return seg_sum(x, edge_flat, zeros)


def _segment_sum_sc2(h, edge_flat):
    n, d = h.shape
    mesh = plsc.VectorSubcoreMesh(core_axis_name="c", subcore_axis_name="s")
    zeros = jnp.zeros((n, d), jnp.float32)

    @functools.partial(
        pl.kernel,
        out_type=jax.ShapeDtypeStruct((NC, n, d), jnp.float32),
        mesh=mesh,
        scratch_types=(
            [pltpu.VMEM((EPW,), jnp.int32)]
            + [pltpu.VMEM((CHUNK,), jnp.int32) for _ in range(NRB)]
            + [pltpu.VMEM((CHUNK, d), jnp.float32) for _ in range(NRB)]
            + [pltpu.VMEM_SHARED((n, d), jnp.float32)]
            + [pltpu.VMEM_SHARED((n, d), jnp.float32)]
            + [pltpu.SemaphoreType.DMA for _ in range(2 * NRB)]
        ),
    )
    def seg_sum2(h_hbm, ei_hbm, zeros_hbm, out_hbm, *refs):
        sidx_all = refs[0]
        didx = refs[1:1 + NRB]
        rows = refs[1 + NRB:1 + 2 * NRB]
        h_spm = refs[1 + 2 * NRB]
        acc = refs[2 + 2 * NRB]
        sems = refs[3 + 2 * NRB:]
        dsem = sems[0:NRB]
        gsem = sems[NRB:2 * NRB]

        cid = lax.axis_index("c")
        sid = lax.axis_index("s")
        wid = sid * NC + cid
        base = wid * EPW

        def start_chunk(g, b):
            pltpu.make_async_copy(
                ei_hbm.at[pl.ds(NW * EPW + base + g * CHUNK, CHUNK)],
                didx[b], dsem[b]
            ).start()
            pltpu.make_async_copy(
                h_spm.at[sidx_all.at[pl.ds(g * CHUNK, CHUNK)]], rows[b], gsem[b]
            ).start()

        @pl.when(sid < NS - 1)
        def _():
            sl = pl.ds(sid * STRIPE, STRIPE)
            pltpu.sync_copy(h_hbm.at[sl], h_spm.at[sl])
            pltpu.sync_copy(zeros_hbm.at[sl], acc.at[sl])

        @pl.when(sid == NS - 1)
        def _():
            sl = pl.ds((NS - 1) * STRIPE, n - (NS - 1) * STRIPE)
            pltpu.sync_copy(h_hbm.at[sl], h_spm.at[sl])
            pltpu.sync_copy(zeros_hbm.at[sl], acc.at[sl])

        pltpu.sync_copy(ei_hbm.at[pl.ds(base, EPW)], sidx_all)
        plsc.subcore_barrier()

        start_chunk(0, 0)
        start_chunk(1, 1)

        nslots = NRB * ((NCHUNKS + NRB - 1) // NRB)

        @pl.loop(0, nslots, step=NRB)
        def _(i):
            for u in range(NRB):
                c = i + u

                @pl.when(c < NCHUNKS)
                def _():
                    pltpu.make_async_copy(
                        ei_hbm.at[pl.ds(base, CHUNK)], didx[u], dsem[u]
                    ).wait()
                    pltpu.make_async_copy(
                        h_hbm.at[pl.ds(0, CHUNK)], rows[u], gsem[u]
                    ).wait()

                    @pl.when(c + 2 < NCHUNKS)
                    def _():
                        start_chunk(c + 2, (u + 2) % NRB)

                    pltpu.sync_copy(rows[u], acc.at[didx[u]], add=True)

        plsc.subcore_barrier()

        @pl.when(sid < NS - 1)
        def _():
            sl = pl.ds(sid * STRIPE, STRIPE)
            pltpu.sync_copy(acc.at[sl], out_hbm.at[cid, sl])

        @pl.when(sid == NS - 1)
        def _():
            sl = pl.ds((NS - 1) * STRIPE, n - (NS - 1) * STRIPE)
            pltpu.sync_copy(acc.at[sl], out_hbm.at[cid, sl])

    return seg_sum2(h, edge_flat, zeros)


def _mlp1_tc(x, parts, W1, b1, W2, b2, eps0):
    n, d_in = x.shape
    h_dim = W1.shape[1]
    blk = 2000

    def body(eps_ref, x_ref, p0_ref, p1_ref, w1_ref, b1_ref, w2_ref, b2_ref,
             o_ref):
        t = (1.0 + eps_ref[0]) * x_ref[...] + p0_ref[0] + p1_ref[0]
        h = jnp.dot(t, w1_ref[...], preferred_element_type=jnp.float32,
                    precision=lax.Precision.HIGHEST) + b1_ref[...]
        h = jnp.maximum(h, 0.0)
        h = jnp.dot(h, w2_ref[...], preferred_element_type=jnp.float32,
                    precision=lax.Precision.HIGHEST) + b2_ref[...]
        h = jnp.maximum(h, 0.0)
        o_ref[...] = jnp.concatenate([h, jnp.zeros_like(h)], axis=1)

    grid = (n // blk,)
    row_spec = pl.BlockSpec((blk, d_in), lambda i: (i, 0))
    part0_spec = pl.BlockSpec((1, blk, d_in), lambda i: (0, i, 0))
    part1_spec = pl.BlockSpec((1, blk, d_in), lambda i: (1, i, 0))
    return pl.pallas_call(
        body,
        grid=grid,
        in_specs=[
            pl.BlockSpec(memory_space=pltpu.SMEM),
            row_spec, part0_spec, part1_spec,
            pl.BlockSpec((d_in, h_dim), lambda i: (0, 0)),
            pl.BlockSpec((1, h_dim), lambda i: (0, 0)),
            pl.BlockSpec((h_dim, h_dim), lambda i: (0, 0)),
            pl.BlockSpec((1, h_dim), lambda i: (0, 0)),
        ],
        out_specs=pl.BlockSpec((blk, 2 * h_dim), lambda i: (i, 0)),
        out_shape=jax.ShapeDtypeStruct((n, 2 * h_dim), jnp.float32),
    )(eps0.reshape(1), x, parts, parts, W1, b1.reshape(1, -1), W2,
      b2.reshape(1, -1))


def _mlp2_tc(h, parts, W3, b3, W4, b4, eps1):
    n, pad_dim = h.shape
    h_dim = W3.shape[0]
    d_out = W4.shape[1]
    blk = 2000

    def body(eps_ref, h_ref, p0_ref, p1_ref, w3_ref, b3_ref, w4_ref, b4_ref,
             o_ref):
        t = (1.0 + eps_ref[0]) * h_ref[...] + p0_ref[0] + p1_ref[0]
        t = t[:, :h_dim]
        g = jnp.dot(t, w3_ref[...], preferred_element_type=jnp.float32,
                    precision=lax.Precision.HIGHEST) + b3_ref[...]
        g = jnp.maximum(g, 0.0)
        logits = jnp.dot(g, w4_ref[...], preferred_element_type=jnp.float32,
                         precision=lax.Precision.HIGHEST) + b4_ref[...]
        m = jnp.max(logits, axis=1, keepdims=True)
        z = logits - m
        lse = jnp.log(jnp.sum(jnp.exp(z), axis=1, keepdims=True))
        o_ref[...] = z - lse

    grid = (n // blk,)
    row_spec = pl.BlockSpec((blk, pad_dim), lambda i: (i, 0))
    part0_spec = pl.BlockSpec((1, blk, pad_dim), lambda i: (0, i, 0))
    part1_spec = pl.BlockSpec((1, blk, pad_dim), lambda i: (1, i, 0))
    return pl.pallas_call(
        body,
        grid=grid,
        in_specs=[
            pl.BlockSpec(memory_space=pltpu.SMEM),
            row_spec, part0_spec, part1_spec,
            pl.BlockSpec((h_dim, h_dim), lambda i: (0, 0)),
            pl.BlockSpec((1, h_dim), lambda i: (0, 0)),
            pl.BlockSpec((h_dim, d_out), lambda i: (0, 0)),
            pl.BlockSpec((1, d_out), lambda i: (0, 0)),
        ],
        out_specs=pl.BlockSpec((blk, d_out), lambda i: (i, 0)),
        out_shape=jax.ShapeDtypeStruct((n, d_out), jnp.float32),
    )(eps1.reshape(1), h, parts, parts, W3, b3.reshape(1, -1), W4,
      b4.reshape(1, -1))


def kernel(x, edge_index, W1, b1, W2, b2, eps0, W3, b3, W4, b4, eps1):
    if PAD:
        srcw = edge_index[0].reshape(NW, EPW_RAW)
        dstw = edge_index[1].reshape(NW, EPW_RAW)
        pad_src = jnp.zeros((NW, PAD), jnp.int32)
        pad_dst = jnp.broadcast_to(
            N_NODES + (jnp.arange(PAD, dtype=jnp.int32) % PAD_ROWS), (NW, PAD))
        src = jnp.concatenate([srcw, pad_src], axis=1).reshape(-1)
        dst = jnp.concatenate([dstw, pad_dst], axis=1).reshape(-1)
        ei = jnp.concatenate([src, dst])
    else:
        ei = edge_index.reshape(-1)

    parts = _segment_sum_sc(x, ei)
    h = _mlp1_tc(x, parts, W1, b1, W2, b2, eps0)

    parts2 = _segment_sum_sc(h, ei)
    return _mlp2_tc(h, parts2, W3, b3, W4, b4, eps1)


# speedup vs baseline: 1.7441x; 1.7441x over previous
import functools

import jax
import jax.numpy as jnp
from jax import lax
from jax.experimental import pallas as pl
from jax.experimental.pallas import tpu as pltpu
from jax.experimental.pallas import tpu_sc as plsc

N_NODES = 10000
N_EDGES = 320000
NC = 2
NS = 16
NW = NC * NS
CHUNK = 80
EPW_RAW = N_EDGES // NW
EPW = 10000
PAD = EPW - EPW_RAW
NCHUNKS = EPW // CHUNK
NRB = 3
PAD_ROWS = 0


STRIPE = 624


def _segment_sum_sc(x, edge_flat):
    n, d = x.shape
    n_acc = n + PAD_ROWS
    mesh = plsc.VectorSubcoreMesh(core_axis_name="c", subcore_axis_name="s")
    zeros = jnp.zeros((n_acc, d), jnp.float32)

    @functools.partial(
        pl.kernel,
        out_type=jax.ShapeDtypeStruct((NC, n, d), jnp.float32),
        mesh=mesh,
        scratch_types=(
            [pltpu.VMEM((EPW,), jnp.int32)]
            + [pltpu.VMEM((CHUNK,), jnp.int32) for _ in range(NRB)]
            + [pltpu.VMEM((CHUNK, d), jnp.float32) for _ in range(NRB)]
            + [pltpu.VMEM_SHARED((n_acc, d), jnp.float32)]
            + [pltpu.SemaphoreType.DMA for _ in range(2 * NRB)]
        ),
    )
    def seg_sum(x_hbm, ei_hbm, zeros_hbm, out_hbm, *refs):
        sidx_all = refs[0]
        didx = refs[1:1 + NRB]
        rows = refs[1 + NRB:1 + 2 * NRB]
        acc = refs[1 + 2 * NRB]
        sems = refs[2 + 2 * NRB:]
        dsem = sems[0:NRB]
        gsem = sems[NRB:2 * NRB]

        cid = lax.axis_index("c")
        sid = lax.axis_index("s")
        wid = sid * NC + cid
        base = wid * EPW

        def start_chunk(g, b):
            pltpu.make_async_copy(
                ei_hbm.at[pl.ds(NW * EPW + base + g * CHUNK, CHUNK)], didx[b], dsem[b]
            ).start()
            pltpu.make_async_copy(
                x_hbm.at[sidx_all.at[pl.ds(g * CHUNK, CHUNK)]], rows[b], gsem[b]
            ).start()

        @pl.when(sid < NS - 1)
        def _():
            pltpu.sync_copy(zeros_hbm.at[pl.ds(sid * STRIPE, STRIPE)],
                            acc.at[pl.ds(sid * STRIPE, STRIPE)])

        @pl.when(sid == NS - 1)
        def _():
            pltpu.sync_copy(zeros_hbm.at[pl.ds((NS - 1) * STRIPE,
                                               n_acc - (NS - 1) * STRIPE)],
                            acc.at[pl.ds((NS - 1) * STRIPE,
                                         n_acc - (NS - 1) * STRIPE)])

        pltpu.sync_copy(ei_hbm.at[pl.ds(base, EPW)], sidx_all)
        plsc.subcore_barrier()

        start_chunk(0, 0)
        start_chunk(1, 1)

        nslots = NRB * ((NCHUNKS + NRB - 1) // NRB)

        @pl.loop(0, nslots, step=NRB)
        def _(i):
            for u in range(NRB):
                c = i + u

                @pl.when(c < NCHUNKS)
                def _():
                    pltpu.make_async_copy(
                        ei_hbm.at[pl.ds(base, CHUNK)], didx[u], dsem[u]
                    ).wait()
                    pltpu.make_async_copy(
                        x_hbm.at[pl.ds(0, CHUNK)], rows[u], gsem[u]
                    ).wait()

                    @pl.when(c + 2 < NCHUNKS)
                    def _():
                        start_chunk(c + 2, (u + 2) % NRB)

                    pltpu.sync_copy(rows[u], acc.at[didx[u]], add=True)

        plsc.subcore_barrier()

        @pl.when(sid < NS - 1)
        def _():
            pltpu.sync_copy(acc.at[pl.ds(sid * STRIPE, STRIPE)],
                            out_hbm.at[cid, pl.ds(sid * STRIPE, STRIPE)])

        @pl.when(sid == NS - 1)
        def _():
            pltpu.sync_copy(acc.at[pl.ds((NS - 1) * STRIPE,
                                         n - (NS - 1) * STRIPE)],
                            out_hbm.at[cid, pl.ds((NS - 1) * STRIPE,
                                                  n - (NS - 1) * STRIPE)])

    return seg_sum(x, edge_flat, zeros)


def _mlp1_tc(x, parts, W1, b1, W2, b2, eps0):
    n, d_in = x.shape
    h_dim = W1.shape[1]
    blk = 2000

    def body(eps_ref, x_ref, p0_ref, p1_ref, w1_ref, b1_ref, w2_ref, b2_ref,
             o_ref):
        t = (1.0 + eps_ref[0]) * x_ref[...] + p0_ref[0] + p1_ref[0]
        h = jnp.dot(t, w1_ref[...], preferred_element_type=jnp.float32,
                    precision=lax.Precision.HIGHEST) + b1_ref[...]
        h = jnp.maximum(h, 0.0)
        h = jnp.dot(h, w2_ref[...], preferred_element_type=jnp.float32,
                    precision=lax.Precision.HIGHEST) + b2_ref[...]
        h = jnp.maximum(h, 0.0)
        o_ref[...] = jnp.concatenate([h, jnp.zeros_like(h)], axis=1)

    grid = (n // blk,)
    row_spec = pl.BlockSpec((blk, d_in), lambda i: (i, 0))
    part0_spec = pl.BlockSpec((1, blk, d_in), lambda i: (0, i, 0))
    part1_spec = pl.BlockSpec((1, blk, d_in), lambda i: (1, i, 0))
    return pl.pallas_call(
        body,
        grid=grid,
        in_specs=[
            pl.BlockSpec(memory_space=pltpu.SMEM),
            row_spec, part0_spec, part1_spec,
            pl.BlockSpec((d_in, h_dim), lambda i: (0, 0)),
            pl.BlockSpec((1, h_dim), lambda i: (0, 0)),
            pl.BlockSpec((h_dim, h_dim), lambda i: (0, 0)),
            pl.BlockSpec((1, h_dim), lambda i: (0, 0)),
        ],
        out_specs=pl.BlockSpec((blk, 2 * h_dim), lambda i: (i, 0)),
        out_shape=jax.ShapeDtypeStruct((n, 2 * h_dim), jnp.float32),
    )(eps0.reshape(1), x, parts, parts, W1, b1.reshape(1, -1), W2,
      b2.reshape(1, -1))


def _mlp2_tc(h, parts, W3, b3, W4, b4, eps1):
    n, pad_dim = h.shape
    h_dim = W3.shape[0]
    d_out = W4.shape[1]
    blk = 2000

    def body(eps_ref, h_ref, p0_ref, p1_ref, w3_ref, b3_ref, w4_ref, b4_ref,
             o_ref):
        t = (1.0 + eps_ref[0]) * h_ref[...] + p0_ref[0] + p1_ref[0]
        t = t[:, :h_dim]
        g = jnp.dot(t, w3_ref[...], preferred_element_type=jnp.float32,
                    precision=lax.Precision.HIGHEST) + b3_ref[...]
        g = jnp.maximum(g, 0.0)
        logits = jnp.dot(g, w4_ref[...], preferred_element_type=jnp.float32,
                         precision=lax.Precision.HIGHEST) + b4_ref[...]
        m = jnp.max(logits, axis=1, keepdims=True)
        z = logits - m
        lse = jnp.log(jnp.sum(jnp.exp(z), axis=1, keepdims=True))
        o_ref[...] = z - lse

    grid = (n // blk,)
    row_spec = pl.BlockSpec((blk, pad_dim), lambda i: (i, 0))
    part0_spec = pl.BlockSpec((1, blk, pad_dim), lambda i: (0, i, 0))
    part1_spec = pl.BlockSpec((1, blk, pad_dim), lambda i: (1, i, 0))
    return pl.pallas_call(
        body,
        grid=grid,
        in_specs=[
            pl.BlockSpec(memory_space=pltpu.SMEM),
            row_spec, part0_spec, part1_spec,
            pl.BlockSpec((h_dim, h_dim), lambda i: (0, 0)),
            pl.BlockSpec((1, h_dim), lambda i: (0, 0)),
            pl.BlockSpec((h_dim, d_out), lambda i: (0, 0)),
            pl.BlockSpec((1, d_out), lambda i: (0, 0)),
        ],
        out_specs=pl.BlockSpec((blk, d_out), lambda i: (i, 0)),
        out_shape=jax.ShapeDtypeStruct((n, d_out), jnp.float32),
    )(eps1.reshape(1), h, parts, parts, W3, b3.reshape(1, -1), W4,
      b4.reshape(1, -1))


def kernel(x, edge_index, W1, b1, W2, b2, eps0, W3, b3, W4, b4, eps1):
    if PAD:
        srcw = edge_index[0].reshape(NW, EPW_RAW)
        dstw = edge_index[1].reshape(NW, EPW_RAW)
        pad_src = jnp.zeros((NW, PAD), jnp.int32)
        pad_dst = jnp.broadcast_to(
            N_NODES + (jnp.arange(PAD, dtype=jnp.int32) % PAD_ROWS), (NW, PAD))
        src = jnp.concatenate([srcw, pad_src], axis=1).reshape(-1)
        dst = jnp.concatenate([dstw, pad_dst], axis=1).reshape(-1)
        ei = jnp.concatenate([src, dst])
    else:
        ei = edge_index.reshape(-1)

    parts = _segment_sum_sc(x, ei)
    h = _mlp1_tc(x, parts, W1, b1, W2, b2, eps0)

    parts2 = _segment_sum_sc(h, ei)
    return _mlp2_tc(h, parts2, W3, b3, W4, b4, eps1)
